# Initial kernel scaffold; baseline (speedup 1.0000x reference)
import jax
import jax.numpy as jnp
from jax.experimental import pallas as pl


def kernel(x, edge_index, params):
    raise NotImplementedError("write your pallas kernel here")


# speedup vs baseline: 5.2484x; 5.2484x over previous
import functools

import numpy as np
import jax
import jax.numpy as jnp
from jax import lax
from jax.experimental import pallas as pl
from jax.experimental.pallas import tpu as pltpu
from jax.experimental.pallas import tpu_sc as plsc

_N = 4096
_E = 65536
_HID = 128
_BN_S = float(1.0 / np.sqrt(1.0 + 1e-5))

_NC = 2
_NS = 16
_NW = _NC * _NS
_EPT = _E // _NW
_CH = 128
_NCHUNK = _EPT // _CH
_RPT = _N // _NS

_sc_mesh = plsc.VectorSubcoreMesh(core_axis_name="c", subcore_axis_name="s")


@functools.partial(
    pl.kernel,
    out_type=jax.ShapeDtypeStruct((_NC, _N, _HID), jnp.float32),
    mesh=_sc_mesh,
    scratch_types=[
        pltpu.VMEM((_NCHUNK, _CH), jnp.int32),
        pltpu.VMEM((_NCHUNK, _CH), jnp.int32),
        pltpu.VMEM((_CH, _HID), jnp.float32),
        pltpu.VMEM_SHARED((_N, _HID), jnp.float32),
        pltpu.SemaphoreType.DMA,
    ],
)
def _sc_scatter_add(h_hbm, src_hbm, dst_hbm, out_hbm,
                    src_v, dst_v, stage_v, acc_sh, sem):
    cid = lax.axis_index("c")
    sid = lax.axis_index("s")
    wid = sid * _NC + cid
    for r in range(16):
        for c in range(_HID // 16):
            stage_v[r, pl.ds(c * 16, 16)] = jnp.zeros((16,), jnp.float32)
    row0 = sid * _RPT
    for i in range(_RPT // 16):
        pltpu.sync_copy(stage_v.at[pl.ds(0, 16)],
                        acc_sh.at[pl.ds(row0 + i * 16, 16)])
    plsc.subcore_barrier()
    pltpu.sync_copy(src_hbm.at[wid], src_v)
    pltpu.sync_copy(dst_hbm.at[wid], dst_v)
    for j in range(_NCHUNK):
        pltpu.async_copy(h_hbm.at[src_v.at[j]], stage_v, sem).wait()
        pltpu.sync_copy(stage_v, acc_sh.at[dst_v.at[j]], add=True)
    plsc.subcore_barrier()
    pltpu.sync_copy(acc_sh.at[pl.ds(row0, _RPT)],
                    out_hbm.at[cid, pl.ds(row0, _RPT)])


_R = 512


def _mlp_body(eps_ref, h_ref, p0_ref, p1_ref, w0_ref, b0_ref, w1_ref, b1_ref,
              o_ref):
    rst = (1.0 + eps_ref[0]) * h_ref[...] + p0_ref[...] + p1_ref[...]
    t = jnp.dot(rst, w0_ref[...], preferred_element_type=jnp.float32)
    t = jnp.maximum(t + b0_ref[...], 0.0)
    t = jnp.dot(t, w1_ref[...], preferred_element_type=jnp.float32)
    o_ref[...] = jnp.maximum(t + b1_ref[...], 0.0)


def _mlp_call(h, p0, p1, eps, w0, b0, w1, b1):
    row = pl.BlockSpec((_R, _HID), lambda i: (i, 0))

    def full(shape):
        return pl.BlockSpec(shape, lambda i, _r=len(shape): (0,) * _r)

    return pl.pallas_call(
        _mlp_body,
        grid=(_N // _R,),
        in_specs=[
            pl.BlockSpec(memory_space=pltpu.SMEM),
            row, row, row,
            full((_HID, _HID)), full((1, _HID)),
            full((_HID, _HID)), full((1, _HID)),
        ],
        out_specs=row,
        out_shape=jax.ShapeDtypeStruct((_N, _HID), jnp.float32),
    )(eps, h, p0, p1, w0, b0, w1, b1)


def _heads_body(x_ref, h1_ref, h2_ref,
                g0_ref, g1_ref, g2_ref, gb_ref,
                c0_ref, c1_ref, c2_ref, cb_ref,
                f0_ref, f1_ref, f2_ref, fb_ref,
                d1_ref, db1_ref, d2_ref, db2_ref,
                gae_ref, cls_ref, fpo_ref):
    x = x_ref[...]
    h1 = h1_ref[...]
    h2 = h2_ref[...]
    dot = functools.partial(jnp.dot, preferred_element_type=jnp.float32)
    gae_ref[...] = (dot(x, g0_ref[...]) + dot(h1, g1_ref[...])
                    + dot(h2, g2_ref[...]) + gb_ref[...])
    p0 = jnp.sum(x, axis=0, keepdims=True)
    p1 = jnp.sum(h1, axis=0, keepdims=True)
    p2 = jnp.sum(h2, axis=0, keepdims=True)
    cls_ref[...] = (dot(p0, c0_ref[...]) + dot(p1, c1_ref[...])
                    + dot(p2, c2_ref[...]) + cb_ref[...])
    fp = (dot(p0, f0_ref[...]) + dot(p1, f1_ref[...])
          + dot(p2, f2_ref[...]) + fb_ref[...])
    t = jnp.maximum(dot(fp, d1_ref[...]) + db1_ref[...], 0.0)
    fpo_ref[...] = dot(t, d2_ref[...]) + db2_ref[...]


def _heads_call(x, h1, h2, gw, gb, cw, cb, fw, fb, d1, db1, d2, db2):
    return pl.pallas_call(
        _heads_body,
        in_specs=[pl.BlockSpec(a.shape, (lambda _r=a.ndim: (0,) * _r))
                  for a in (x, h1, h2, gw[0], gw[1], gw[2], gb,
                            cw[0], cw[1], cw[2], cb,
                            fw[0], fw[1], fw[2], fb, d1, db1, d2, db2)],
        out_specs=[
            pl.BlockSpec((_N, 2 * _HID), lambda: (0, 0)),
            pl.BlockSpec((1, 16), lambda: (0, 0)),
            pl.BlockSpec((1, 2048), lambda: (0, 0)),
        ],
        out_shape=[
            jax.ShapeDtypeStruct((_N, 2 * _HID), jnp.float32),
            jax.ShapeDtypeStruct((1, 16), jnp.float32),
            jax.ShapeDtypeStruct((1, 2048), jnp.float32),
        ],
    )(x, h1, h2, gw[0], gw[1], gw[2], gb, cw[0], cw[1], cw[2], cb,
      fw[0], fw[1], fw[2], fb, d1, db1, d2, db2)


def _adj_body(mu_i_ref, mu_ref, o_ref):
    o_ref[...] = lax.dot_general(
        mu_i_ref[...], mu_ref[...], (((1,), (1,)), ((), ())),
        preferred_element_type=jnp.float32)


def _adj_call(mu):
    return pl.pallas_call(
        _adj_body,
        grid=(_N // _R,),
        in_specs=[
            pl.BlockSpec((_R, _HID), lambda i: (i, 0)),
            pl.BlockSpec((_N, _HID), lambda i: (0, 0)),
        ],
        out_specs=pl.BlockSpec((_R, _N), lambda i: (i, 0)),
        out_shape=jax.ShapeDtypeStruct((_N, _N), jnp.float32),
    )(mu, mu)


def kernel(x, edge_index, params):
    src = edge_index[0].reshape(_NW, _NCHUNK, _CH)
    dst = edge_index[1].reshape(_NW, _NCHUNK, _CH)

    def fold(w, b, g, bb):
        scale = g * _BN_S
        return w * scale[None, :], (b * scale + bb)[None, :]

    h = x
    hidden = [x]
    for l in range(2):
        parts = _sc_scatter_add(h, src, dst)
        w0, b0 = fold(params[f"gin{l}_W0"], params[f"gin{l}_b0"],
                      params[f"gin{l}_bn0_g"], params[f"gin{l}_bn0_b"])
        w1, b1 = fold(params[f"gin{l}_W1"], params[f"gin{l}_b1"],
                      params[f"gin{l}_bna_g"], params[f"gin{l}_bna_b"])
        eps = params[f"gin{l}_eps"].reshape(1)
        h = _mlp_call(h, parts[0], parts[1], eps, w0, b0, w1, b1)
        hidden.append(h)

    gw = [params[f"gae{i}_W"] for i in range(3)]
    gb = (params["gae0_b"] + params["gae1_b"] + params["gae2_b"])[None, :]
    cw = [params[f"cls{i}_W"] for i in range(3)]
    cb = (params["cls0_b"] + params["cls1_b"] + params["cls2_b"])[None, :]
    fw = [params[f"fp{i}_W"] for i in range(3)]
    fb = (params["fp0_b"] + params["fp1_b"] + params["fp2_b"])[None, :]
    gae, cls, fpo = _heads_call(
        hidden[0], hidden[1], hidden[2], gw, gb, cw, cb, fw, fb,
        params["fd_W1"], params["fd_b1"][None, :],
        params["fd_W2"], params["fd_b2"][None, :])
    mu = gae[:, :_HID]
    logvar = gae[:, _HID:]
    adj = _adj_call(mu)
    return adj, mu, logvar, cls, fpo


# speedup vs baseline: 7.2304x; 1.3777x over previous
import functools

import numpy as np
import jax
import jax.numpy as jnp
from jax import lax
from jax.experimental import pallas as pl
from jax.experimental.pallas import tpu as pltpu
from jax.experimental.pallas import tpu_sc as plsc

_N = 4096
_E = 65536
_HID = 128
_BN_S = float(1.0 / np.sqrt(1.0 + 1e-5))

_NC = 2
_NS = 16
_NW = _NC * _NS
_EPT = _E // _NW
_CH = 128
_NCHUNK = _EPT // _CH
_RPT = _N // _NS

_sc_mesh = plsc.VectorSubcoreMesh(core_axis_name="c", subcore_axis_name="s")


_NBUF = 4


@functools.partial(
    pl.kernel,
    out_type=jax.ShapeDtypeStruct((_NC, _N, _HID), jnp.float32),
    mesh=_sc_mesh,
    scratch_types=[
        pltpu.VMEM((_NCHUNK, _CH), jnp.int32),
        pltpu.VMEM((_NCHUNK, _CH), jnp.int32),
        pltpu.VMEM((_NBUF, _CH, _HID), jnp.float32),
        pltpu.VMEM_SHARED((_N, _HID), jnp.float32),
    ] + [pltpu.SemaphoreType.DMA] * (2 * _NBUF),
)
def _sc_scatter_add(h_hbm, src_hbm, dst_hbm, out_hbm,
                    src_v, dst_v, stage_v, acc_sh, *sems):
    gsems = sems[:_NBUF]
    ssems = sems[_NBUF:]
    cid = lax.axis_index("c")
    sid = lax.axis_index("s")
    wid = sid * _NC + cid
    gidx = pltpu.async_copy(src_hbm.at[wid], src_v, gsems[0])
    didx = pltpu.async_copy(dst_hbm.at[wid], dst_v, gsems[1])
    for r in range(16):
        for c in range(_HID // 16):
            stage_v[0, r, pl.ds(c * 16, 16)] = jnp.zeros((16,), jnp.float32)
    row0 = sid * _RPT
    for i in range(_RPT // 16):
        pltpu.sync_copy(stage_v.at[0, pl.ds(0, 16)],
                        acc_sh.at[pl.ds(row0 + i * 16, 16)])
    gidx.wait()
    didx.wait()
    plsc.subcore_barrier()
    gd = {}
    sd = {}
    for j in range(_NBUF):
        gd[j] = pltpu.async_copy(h_hbm.at[src_v.at[j]], stage_v.at[j],
                                 gsems[j])
    for j in range(_NCHUNK):
        b = j % _NBUF
        gd[j].wait()
        sd[j] = pltpu.async_copy(stage_v.at[b], acc_sh.at[dst_v.at[j]],
                                 ssems[b], add=True)
        if j + _NBUF < _NCHUNK:
            sd[j].wait()
            gd[j + _NBUF] = pltpu.async_copy(
                h_hbm.at[src_v.at[j + _NBUF]], stage_v.at[b], gsems[b])
    for j in range(_NCHUNK - _NBUF, _NCHUNK):
        sd[j].wait()
    plsc.subcore_barrier()
    pltpu.sync_copy(acc_sh.at[pl.ds(row0, _RPT)],
                    out_hbm.at[cid, pl.ds(row0, _RPT)])


_R = 512


def _mlp_body(eps_ref, h_ref, p_ref, w0_ref, b0_ref, w1_ref, b1_ref,
              o_ref):
    rst = (1.0 + eps_ref[0]) * h_ref[...] + p_ref[0] + p_ref[1]
    t = jnp.dot(rst, w0_ref[...], preferred_element_type=jnp.float32)
    t = jnp.maximum(t + b0_ref[...], 0.0)
    t = jnp.dot(t, w1_ref[...], preferred_element_type=jnp.float32)
    o_ref[...] = jnp.maximum(t + b1_ref[...], 0.0)


def _mlp_call(h, parts, eps, w0, b0, w1, b1):
    row = pl.BlockSpec((_R, _HID), lambda i: (i, 0))

    def full(shape):
        return pl.BlockSpec(shape, lambda i, _r=len(shape): (0,) * _r)

    return pl.pallas_call(
        _mlp_body,
        grid=(_N // _R,),
        in_specs=[
            pl.BlockSpec(memory_space=pltpu.SMEM),
            row,
            pl.BlockSpec((_NC, _R, _HID), lambda i: (0, i, 0)),
            full((_HID, _HID)), full((1, _HID)),
            full((_HID, _HID)), full((1, _HID)),
        ],
        out_specs=row,
        out_shape=jax.ShapeDtypeStruct((_N, _HID), jnp.float32),
    )(eps, h, parts, w0, b0, w1, b1)


def _heads_body(x_ref, h1_ref, h2_ref,
                g0_ref, g1_ref, g2_ref, gb_ref,
                c0_ref, c1_ref, c2_ref, cb_ref,
                f0_ref, f1_ref, f2_ref, fb_ref,
                d1_ref, db1_ref, d2_ref, db2_ref,
                mu_ref, lv_ref, cls_ref, fpo_ref):
    x = x_ref[...]
    h1 = h1_ref[...]
    h2 = h2_ref[...]
    dot = functools.partial(jnp.dot, preferred_element_type=jnp.float32)
    gae = (dot(x, g0_ref[...]) + dot(h1, g1_ref[...])
           + dot(h2, g2_ref[...]) + gb_ref[...])
    mu_ref[...] = gae[:, :_HID]
    lv_ref[...] = gae[:, _HID:]
    p0 = jnp.sum(x, axis=0, keepdims=True)
    p1 = jnp.sum(h1, axis=0, keepdims=True)
    p2 = jnp.sum(h2, axis=0, keepdims=True)
    cls_ref[...] = (dot(p0, c0_ref[...]) + dot(p1, c1_ref[...])
                    + dot(p2, c2_ref[...]) + cb_ref[...])
    fp = (dot(p0, f0_ref[...]) + dot(p1, f1_ref[...])
          + dot(p2, f2_ref[...]) + fb_ref[...])
    t = jnp.maximum(dot(fp, d1_ref[...]) + db1_ref[...], 0.0)
    fpo_ref[...] = dot(t, d2_ref[...]) + db2_ref[...]


def _heads_call(x, h1, h2, gw, gb, cw, cb, fw, fb, d1, db1, d2, db2):
    return pl.pallas_call(
        _heads_body,
        in_specs=[pl.BlockSpec(a.shape, (lambda _r=a.ndim: (0,) * _r))
                  for a in (x, h1, h2, gw[0], gw[1], gw[2], gb,
                            cw[0], cw[1], cw[2], cb,
                            fw[0], fw[1], fw[2], fb, d1, db1, d2, db2)],
        out_specs=[
            pl.BlockSpec((_N, _HID), lambda: (0, 0)),
            pl.BlockSpec((_N, _HID), lambda: (0, 0)),
            pl.BlockSpec((1, 16), lambda: (0, 0)),
            pl.BlockSpec((1, 2048), lambda: (0, 0)),
        ],
        out_shape=[
            jax.ShapeDtypeStruct((_N, _HID), jnp.float32),
            jax.ShapeDtypeStruct((_N, _HID), jnp.float32),
            jax.ShapeDtypeStruct((1, 16), jnp.float32),
            jax.ShapeDtypeStruct((1, 2048), jnp.float32),
        ],
    )(x, h1, h2, gw[0], gw[1], gw[2], gb, cw[0], cw[1], cw[2], cb,
      fw[0], fw[1], fw[2], fb, d1, db1, d2, db2)


def _adj_body(mu_i_ref, mu_ref, o_ref):
    o_ref[...] = lax.dot_general(
        mu_i_ref[...], mu_ref[...], (((1,), (1,)), ((), ())),
        preferred_element_type=jnp.float32)


def _adj_call(mu):
    return pl.pallas_call(
        _adj_body,
        grid=(_N // _R,),
        in_specs=[
            pl.BlockSpec((_R, _HID), lambda i: (i, 0)),
            pl.BlockSpec((_N, _HID), lambda i: (0, 0)),
        ],
        out_specs=pl.BlockSpec((_R, _N), lambda i: (i, 0)),
        out_shape=jax.ShapeDtypeStruct((_N, _N), jnp.float32),
    )(mu, mu)


def kernel(x, edge_index, params):
    src = edge_index[0].reshape(_NW, _NCHUNK, _CH)
    dst = edge_index[1].reshape(_NW, _NCHUNK, _CH)

    def fold(w, b, g, bb):
        scale = g * _BN_S
        return w * scale[None, :], (b * scale + bb)[None, :]

    h = x
    hidden = [x]
    for l in range(2):
        parts = _sc_scatter_add(h, src, dst)
        w0, b0 = fold(params[f"gin{l}_W0"], params[f"gin{l}_b0"],
                      params[f"gin{l}_bn0_g"], params[f"gin{l}_bn0_b"])
        w1, b1 = fold(params[f"gin{l}_W1"], params[f"gin{l}_b1"],
                      params[f"gin{l}_bna_g"], params[f"gin{l}_bna_b"])
        eps = params[f"gin{l}_eps"].reshape(1)
        h = _mlp_call(h, parts, eps, w0, b0, w1, b1)
        hidden.append(h)

    gw = [params[f"gae{i}_W"] for i in range(3)]
    gb = (params["gae0_b"] + params["gae1_b"] + params["gae2_b"])[None, :]
    cw = [params[f"cls{i}_W"] for i in range(3)]
    cb = (params["cls0_b"] + params["cls1_b"] + params["cls2_b"])[None, :]
    fw = [params[f"fp{i}_W"] for i in range(3)]
    fb = (params["fp0_b"] + params["fp1_b"] + params["fp2_b"])[None, :]
    mu, logvar, cls, fpo = _heads_call(
        hidden[0], hidden[1], hidden[2], gw, gb, cw, cb, fw, fb,
        params["fd_W1"], params["fd_b1"][None, :],
        params["fd_W2"], params["fd_b2"][None, :])
    adj = _adj_call(mu)
    return adj, mu, logvar, cls, fpo


# speedup vs baseline: 7.4238x; 1.0267x over previous
import functools

import numpy as np
import jax
import jax.numpy as jnp
from jax import lax
from jax.experimental import pallas as pl
from jax.experimental.pallas import tpu as pltpu
from jax.experimental.pallas import tpu_sc as plsc

_N = 4096
_E = 65536
_HID = 128
_BN_S = float(1.0 / np.sqrt(1.0 + 1e-5))

_NC = 2
_NS = 16
_NW = _NC * _NS
_EPT = _E // _NW
_CH = 128
_NCHUNK = _EPT // _CH
_RPT = _N // _NS

_sc_mesh = plsc.VectorSubcoreMesh(core_axis_name="c", subcore_axis_name="s")


_NBUF = 5


@functools.partial(
    pl.kernel,
    out_type=jax.ShapeDtypeStruct((_NC, _N, _HID), jnp.float32),
    mesh=_sc_mesh,
    scratch_types=[
        pltpu.VMEM((_NCHUNK, _CH), jnp.int32),
        pltpu.VMEM((_NCHUNK, _CH), jnp.int32),
        pltpu.VMEM((_NBUF, _CH, _HID), jnp.float32),
        pltpu.VMEM_SHARED((_N, _HID), jnp.float32),
    ] + [pltpu.SemaphoreType.DMA] * (2 * _NBUF),
)
def _sc_scatter_add(h_hbm, src_hbm, dst_hbm, out_hbm,
                    src_v, dst_v, stage_v, acc_sh, *sems):
    gsems = sems[:_NBUF]
    ssems = sems[_NBUF:]
    cid = lax.axis_index("c")
    sid = lax.axis_index("s")
    wid = sid * _NC + cid
    gidx = pltpu.async_copy(src_hbm.at[wid], src_v, gsems[0])
    didx = pltpu.async_copy(dst_hbm.at[wid], dst_v, gsems[1])
    for r in range(16):
        for c in range(_HID // 16):
            stage_v[0, r, pl.ds(c * 16, 16)] = jnp.zeros((16,), jnp.float32)
    row0 = sid * _RPT
    for i in range(_RPT // 16):
        pltpu.sync_copy(stage_v.at[0, pl.ds(0, 16)],
                        acc_sh.at[pl.ds(row0 + i * 16, 16)])
    gidx.wait()
    didx.wait()
    plsc.subcore_barrier()
    gd = {}
    sd = {}
    for j in range(_NBUF):
        gd[j] = pltpu.async_copy(h_hbm.at[src_v.at[j]], stage_v.at[j],
                                 gsems[j])
    for j in range(_NCHUNK):
        b = j % _NBUF
        gd[j].wait()
        sd[j] = pltpu.async_copy(stage_v.at[b], acc_sh.at[dst_v.at[j]],
                                 ssems[b], add=True)
        if j + _NBUF < _NCHUNK:
            sd[j].wait()
            gd[j + _NBUF] = pltpu.async_copy(
                h_hbm.at[src_v.at[j + _NBUF]], stage_v.at[b], gsems[b])
    for j in range(_NCHUNK - _NBUF, _NCHUNK):
        sd[j].wait()
    plsc.subcore_barrier()
    pltpu.sync_copy(acc_sh.at[pl.ds(row0, _RPT)],
                    out_hbm.at[cid, pl.ds(row0, _RPT)])


_R = 512


def _mlp_body(eps_ref, h_ref, p_ref, w0_ref, b0_ref, w1_ref, b1_ref,
              o_ref):
    rst = (1.0 + eps_ref[0]) * h_ref[...] + p_ref[0] + p_ref[1]
    t = jnp.dot(rst, w0_ref[...], preferred_element_type=jnp.float32)
    t = jnp.maximum(t + b0_ref[...], 0.0)
    t = jnp.dot(t, w1_ref[...], preferred_element_type=jnp.float32)
    o_ref[...] = jnp.maximum(t + b1_ref[...], 0.0)


def _mlp_call(h, parts, eps, w0, b0, w1, b1):
    row = pl.BlockSpec((_R, _HID), lambda i: (i, 0))

    def full(shape):
        return pl.BlockSpec(shape, lambda i, _r=len(shape): (0,) * _r)

    return pl.pallas_call(
        _mlp_body,
        grid=(_N // _R,),
        in_specs=[
            pl.BlockSpec(memory_space=pltpu.SMEM),
            row,
            pl.BlockSpec((_NC, _R, _HID), lambda i: (0, i, 0)),
            full((_HID, _HID)), full((1, _HID)),
            full((_HID, _HID)), full((1, _HID)),
        ],
        out_specs=row,
        out_shape=jax.ShapeDtypeStruct((_N, _HID), jnp.float32),
    )(eps, h, parts, w0, b0, w1, b1)


def _mlp1_heads_body(eps_ref, x_ref, h1_ref, p_ref, w0_ref, b0_ref,
                     w1_ref, b1_ref,
                     g0_ref, g1_ref, g2_ref, gb_ref,
                     c0_ref, c1_ref, c2_ref, cb_ref,
                     f0_ref, f1_ref, f2_ref, fb_ref,
                     d1_ref, db1_ref, d2_ref, db2_ref,
                     mu_ref, lv_ref, cls_ref, fpo_ref,
                     a0_ref, a1_ref, a2_ref):
    i = pl.program_id(0)
    dot = functools.partial(jnp.dot, preferred_element_type=jnp.float32)
    x = x_ref[...]
    h1 = h1_ref[...]
    rst = (1.0 + eps_ref[0]) * h1 + p_ref[0] + p_ref[1]
    t = jnp.maximum(dot(rst, w0_ref[...]) + b0_ref[...], 0.0)
    h2 = jnp.maximum(dot(t, w1_ref[...]) + b1_ref[...], 0.0)
    gae = (dot(x, g0_ref[...]) + dot(h1, g1_ref[...])
           + dot(h2, g2_ref[...]) + gb_ref[...])
    mu_ref[...] = gae[:, :_HID]
    lv_ref[...] = gae[:, _HID:]

    s0 = jnp.sum(x, axis=0, keepdims=True)
    s1 = jnp.sum(h1, axis=0, keepdims=True)
    s2 = jnp.sum(h2, axis=0, keepdims=True)

    @pl.when(i == 0)
    def _():
        a0_ref[...] = s0
        a1_ref[...] = s1
        a2_ref[...] = s2

    @pl.when(i > 0)
    def _():
        a0_ref[...] += s0
        a1_ref[...] += s1
        a2_ref[...] += s2

    @pl.when(i == _N // _R - 1)
    def _():
        p0 = a0_ref[...]
        p1 = a1_ref[...]
        p2 = a2_ref[...]
        cls_ref[...] = (dot(p0, c0_ref[...]) + dot(p1, c1_ref[...])
                        + dot(p2, c2_ref[...]) + cb_ref[...])
        fp = (dot(p0, f0_ref[...]) + dot(p1, f1_ref[...])
              + dot(p2, f2_ref[...]) + fb_ref[...])
        tt = jnp.maximum(dot(fp, d1_ref[...]) + db1_ref[...], 0.0)
        fpo_ref[...] = dot(tt, d2_ref[...]) + db2_ref[...]


def _mlp1_heads_call(x, h1, parts, eps, w0, b0, w1, b1,
                     gw, gb, cw, cb, fw, fb, d1, db1, d2, db2):
    row = pl.BlockSpec((_R, _HID), lambda i: (i, 0))

    def full(a):
        return pl.BlockSpec(a.shape, (lambda i, _r=a.ndim: (0,) * _r))

    return pl.pallas_call(
        _mlp1_heads_body,
        grid=(_N // _R,),
        in_specs=[pl.BlockSpec(memory_space=pltpu.SMEM),
                  row, row,
                  pl.BlockSpec((_NC, _R, _HID), lambda i: (0, i, 0))] +
                 [full(a) for a in (w0, b0, w1, b1,
                                    gw[0], gw[1], gw[2], gb,
                                    cw[0], cw[1], cw[2], cb,
                                    fw[0], fw[1], fw[2], fb,
                                    d1, db1, d2, db2)],
        out_specs=[
            row, row,
            pl.BlockSpec((1, 16), lambda i: (0, 0)),
            pl.BlockSpec((1, 2048), lambda i: (0, 0)),
        ],
        out_shape=[
            jax.ShapeDtypeStruct((_N, _HID), jnp.float32),
            jax.ShapeDtypeStruct((_N, _HID), jnp.float32),
            jax.ShapeDtypeStruct((1, 16), jnp.float32),
            jax.ShapeDtypeStruct((1, 2048), jnp.float32),
        ],
        scratch_shapes=[pltpu.VMEM((1, _HID), jnp.float32)] * 3,
    )(eps, x, h1, parts, w0, b0, w1, b1,
      gw[0], gw[1], gw[2], gb, cw[0], cw[1], cw[2], cb,
      fw[0], fw[1], fw[2], fb, d1, db1, d2, db2)


def _adj_body(mu_i_ref, mu_ref, o_ref):
    o_ref[...] = lax.dot_general(
        mu_i_ref[...], mu_ref[...], (((1,), (1,)), ((), ())),
        preferred_element_type=jnp.float32)


def _adj_call(mu):
    return pl.pallas_call(
        _adj_body,
        grid=(_N // _R,),
        in_specs=[
            pl.BlockSpec((_R, _HID), lambda i: (i, 0)),
            pl.BlockSpec((_N, _HID), lambda i: (0, 0)),
        ],
        out_specs=pl.BlockSpec((_R, _N), lambda i: (i, 0)),
        out_shape=jax.ShapeDtypeStruct((_N, _N), jnp.float32),
    )(mu, mu)


def kernel(x, edge_index, params):
    src = edge_index[0].reshape(_NW, _NCHUNK, _CH)
    dst = edge_index[1].reshape(_NW, _NCHUNK, _CH)

    def fold(w, b, g, bb):
        scale = g * _BN_S
        return w * scale[None, :], (b * scale + bb)[None, :]

    def layer_weights(l):
        w0, b0 = fold(params[f"gin{l}_W0"], params[f"gin{l}_b0"],
                      params[f"gin{l}_bn0_g"], params[f"gin{l}_bn0_b"])
        w1, b1 = fold(params[f"gin{l}_W1"], params[f"gin{l}_b1"],
                      params[f"gin{l}_bna_g"], params[f"gin{l}_bna_b"])
        return w0, b0, w1, b1, params[f"gin{l}_eps"].reshape(1)

    parts0 = _sc_scatter_add(x, src, dst)
    w0, b0, w1, b1, eps0 = layer_weights(0)
    h1 = _mlp_call(x, parts0, eps0, w0, b0, w1, b1)

    parts1 = _sc_scatter_add(h1, src, dst)
    w0, b0, w1, b1, eps1 = layer_weights(1)

    gw = [params[f"gae{i}_W"] for i in range(3)]
    gb = (params["gae0_b"] + params["gae1_b"] + params["gae2_b"])[None, :]
    cw = [params[f"cls{i}_W"] for i in range(3)]
    cb = (params["cls0_b"] + params["cls1_b"] + params["cls2_b"])[None, :]
    fw = [params[f"fp{i}_W"] for i in range(3)]
    fb = (params["fp0_b"] + params["fp1_b"] + params["fp2_b"])[None, :]
    mu, logvar, cls, fpo = _mlp1_heads_call(
        x, h1, parts1, eps1, w0, b0, w1, b1, gw, gb, cw, cb, fw, fb,
        params["fd_W1"], params["fd_b1"][None, :],
        params["fd_W2"], params["fd_b2"][None, :])
    adj = _adj_call(mu)
    return adj, mu, logvar, cls, fpo


# speedup vs baseline: 7.4850x; 1.0083x over previous
import functools

import numpy as np
import jax
import jax.numpy as jnp
from jax import lax
from jax.experimental import pallas as pl
from jax.experimental.pallas import tpu as pltpu
from jax.experimental.pallas import tpu_sc as plsc

_N = 4096
_E = 65536
_HID = 128
_BN_S = float(1.0 / np.sqrt(1.0 + 1e-5))

_NC = 2
_NS = 16
_NW = _NC * _NS
_EPT = _E // _NW
_CH = 128
_NCHUNK = _EPT // _CH
_RPT = _N // _NS

_sc_mesh = plsc.VectorSubcoreMesh(core_axis_name="c", subcore_axis_name="s")


_NBUF = 4


@functools.partial(
    pl.kernel,
    out_type=jax.ShapeDtypeStruct((_NC, _N, _HID), jnp.float32),
    mesh=_sc_mesh,
    scratch_types=[
        pltpu.VMEM((_NCHUNK, _CH), jnp.int32),
        pltpu.VMEM((_NCHUNK, _CH), jnp.int32),
        pltpu.VMEM((_NBUF, _CH, _HID), jnp.float32),
        pltpu.VMEM_SHARED((_N, _HID), jnp.float32),
    ] + [pltpu.SemaphoreType.DMA] * _NBUF,
)
def _sc_scatter_add(h_hbm, src_hbm, dst_hbm, out_hbm,
                    src_v, dst_v, stage_v, acc_sh, *gsems):
    cid = lax.axis_index("c")
    sid = lax.axis_index("s")
    wid = sid * _NC + cid
    gidx = pltpu.async_copy(src_hbm.at[wid], src_v, gsems[0])
    didx = pltpu.async_copy(dst_hbm.at[wid], dst_v, gsems[1])
    for r in range(16):
        for c in range(_HID // 16):
            stage_v[0, r, pl.ds(c * 16, 16)] = jnp.zeros((16,), jnp.float32)
    row0 = sid * _RPT
    for i in range(_RPT // 16):
        pltpu.sync_copy(stage_v.at[0, pl.ds(0, 16)],
                        acc_sh.at[pl.ds(row0 + i * 16, 16)])
    gidx.wait()
    didx.wait()
    plsc.subcore_barrier()
    gd = {}
    for j in range(_NBUF):
        gd[j] = pltpu.async_copy(h_hbm.at[src_v.at[j]], stage_v.at[j],
                                 gsems[j])
    for j in range(_NCHUNK):
        b = j % _NBUF
        gd[j].wait()
        pltpu.sync_copy(stage_v.at[b], acc_sh.at[dst_v.at[j]], add=True)
        if j + _NBUF < _NCHUNK:
            gd[j + _NBUF] = pltpu.async_copy(
                h_hbm.at[src_v.at[j + _NBUF]], stage_v.at[b], gsems[b])
    plsc.subcore_barrier()
    pltpu.sync_copy(acc_sh.at[pl.ds(row0, _RPT)],
                    out_hbm.at[cid, pl.ds(row0, _RPT)])


_R = 512


def _mlp_body(eps_ref, h_ref, p_ref, w0_ref, b0_ref, w1_ref, b1_ref,
              o_ref):
    rst = (1.0 + eps_ref[0]) * h_ref[...] + p_ref[0] + p_ref[1]
    t = jnp.dot(rst, w0_ref[...], preferred_element_type=jnp.float32)
    t = jnp.maximum(t + b0_ref[...], 0.0)
    t = jnp.dot(t, w1_ref[...], preferred_element_type=jnp.float32)
    o_ref[...] = jnp.maximum(t + b1_ref[...], 0.0)


def _mlp_call(h, parts, eps, w0, b0, w1, b1):
    row = pl.BlockSpec((_R, _HID), lambda i: (i, 0))

    def full(shape):
        return pl.BlockSpec(shape, lambda i, _r=len(shape): (0,) * _r)

    return pl.pallas_call(
        _mlp_body,
        grid=(_N // _R,),
        in_specs=[
            pl.BlockSpec(memory_space=pltpu.SMEM),
            row,
            pl.BlockSpec((_NC, _R, _HID), lambda i: (0, i, 0)),
            full((_HID, _HID)), full((1, _HID)),
            full((_HID, _HID)), full((1, _HID)),
        ],
        out_specs=row,
        out_shape=jax.ShapeDtypeStruct((_N, _HID), jnp.float32),
    )(eps, h, parts, w0, b0, w1, b1)


def _mlp1_heads_body(eps_ref, x_ref, h1_ref, p_ref, w0_ref, b0_ref,
                     w1_ref, b1_ref,
                     g0_ref, g1_ref, g2_ref, gb_ref,
                     c0_ref, c1_ref, c2_ref, cb_ref,
                     f0_ref, f1_ref, f2_ref, fb_ref,
                     d1_ref, db1_ref, d2_ref, db2_ref,
                     mu_ref, lv_ref, cls_ref, fpo_ref,
                     a0_ref, a1_ref, a2_ref):
    i = pl.program_id(0)
    dot = functools.partial(jnp.dot, preferred_element_type=jnp.float32)
    x = x_ref[...]
    h1 = h1_ref[...]
    rst = (1.0 + eps_ref[0]) * h1 + p_ref[0] + p_ref[1]
    t = jnp.maximum(dot(rst, w0_ref[...]) + b0_ref[...], 0.0)
    h2 = jnp.maximum(dot(t, w1_ref[...]) + b1_ref[...], 0.0)
    gae = (dot(x, g0_ref[...]) + dot(h1, g1_ref[...])
           + dot(h2, g2_ref[...]) + gb_ref[...])
    mu_ref[...] = gae[:, :_HID]
    lv_ref[...] = gae[:, _HID:]

    s0 = jnp.sum(x, axis=0, keepdims=True)
    s1 = jnp.sum(h1, axis=0, keepdims=True)
    s2 = jnp.sum(h2, axis=0, keepdims=True)

    @pl.when(i == 0)
    def _():
        a0_ref[...] = s0
        a1_ref[...] = s1
        a2_ref[...] = s2

    @pl.when(i > 0)
    def _():
        a0_ref[...] += s0
        a1_ref[...] += s1
        a2_ref[...] += s2

    @pl.when(i == _N // _R - 1)
    def _():
        p0 = a0_ref[...]
        p1 = a1_ref[...]
        p2 = a2_ref[...]
        cls_ref[...] = (dot(p0, c0_ref[...]) + dot(p1, c1_ref[...])
                        + dot(p2, c2_ref[...]) + cb_ref[...])
        fp = (dot(p0, f0_ref[...]) + dot(p1, f1_ref[...])
              + dot(p2, f2_ref[...]) + fb_ref[...])
        tt = jnp.maximum(dot(fp, d1_ref[...]) + db1_ref[...], 0.0)
        fpo_ref[...] = dot(tt, d2_ref[...]) + db2_ref[...]


def _mlp1_heads_call(x, h1, parts, eps, w0, b0, w1, b1,
                     gw, gb, cw, cb, fw, fb, d1, db1, d2, db2):
    row = pl.BlockSpec((_R, _HID), lambda i: (i, 0))

    def full(a):
        return pl.BlockSpec(a.shape, (lambda i, _r=a.ndim: (0,) * _r))

    return pl.pallas_call(
        _mlp1_heads_body,
        grid=(_N // _R,),
        in_specs=[pl.BlockSpec(memory_space=pltpu.SMEM),
                  row, row,
                  pl.BlockSpec((_NC, _R, _HID), lambda i: (0, i, 0))] +
                 [full(a) for a in (w0, b0, w1, b1,
                                    gw[0], gw[1], gw[2], gb,
                                    cw[0], cw[1], cw[2], cb,
                                    fw[0], fw[1], fw[2], fb,
                                    d1, db1, d2, db2)],
        out_specs=[
            row, row,
            pl.BlockSpec((1, 16), lambda i: (0, 0)),
            pl.BlockSpec((1, 2048), lambda i: (0, 0)),
        ],
        out_shape=[
            jax.ShapeDtypeStruct((_N, _HID), jnp.float32),
            jax.ShapeDtypeStruct((_N, _HID), jnp.float32),
            jax.ShapeDtypeStruct((1, 16), jnp.float32),
            jax.ShapeDtypeStruct((1, 2048), jnp.float32),
        ],
        scratch_shapes=[pltpu.VMEM((1, _HID), jnp.float32)] * 3,
    )(eps, x, h1, parts, w0, b0, w1, b1,
      gw[0], gw[1], gw[2], gb, cw[0], cw[1], cw[2], cb,
      fw[0], fw[1], fw[2], fb, d1, db1, d2, db2)


def _adj_body(mu_i_ref, mu_ref, o_ref):
    o_ref[...] = lax.dot_general(
        mu_i_ref[...], mu_ref[...], (((1,), (1,)), ((), ())),
        preferred_element_type=jnp.float32)


def _adj_call(mu):
    return pl.pallas_call(
        _adj_body,
        grid=(_N // _R,),
        in_specs=[
            pl.BlockSpec((_R, _HID), lambda i: (i, 0)),
            pl.BlockSpec((_N, _HID), lambda i: (0, 0)),
        ],
        out_specs=pl.BlockSpec((_R, _N), lambda i: (i, 0)),
        out_shape=jax.ShapeDtypeStruct((_N, _N), jnp.float32),
    )(mu, mu)


def kernel(x, edge_index, params):
    src = edge_index[0].reshape(_NW, _NCHUNK, _CH)
    dst = edge_index[1].reshape(_NW, _NCHUNK, _CH)

    def fold(w, b, g, bb):
        scale = g * _BN_S
        return w * scale[None, :], (b * scale + bb)[None, :]

    def layer_weights(l):
        w0, b0 = fold(params[f"gin{l}_W0"], params[f"gin{l}_b0"],
                      params[f"gin{l}_bn0_g"], params[f"gin{l}_bn0_b"])
        w1, b1 = fold(params[f"gin{l}_W1"], params[f"gin{l}_b1"],
                      params[f"gin{l}_bna_g"], params[f"gin{l}_bna_b"])
        return w0, b0, w1, b1, params[f"gin{l}_eps"].reshape(1)

    parts0 = _sc_scatter_add(x, src, dst)
    w0, b0, w1, b1, eps0 = layer_weights(0)
    h1 = _mlp_call(x, parts0, eps0, w0, b0, w1, b1)

    parts1 = _sc_scatter_add(h1, src, dst)
    w0, b0, w1, b1, eps1 = layer_weights(1)

    gw = [params[f"gae{i}_W"] for i in range(3)]
    gb = (params["gae0_b"] + params["gae1_b"] + params["gae2_b"])[None, :]
    cw = [params[f"cls{i}_W"] for i in range(3)]
    cb = (params["cls0_b"] + params["cls1_b"] + params["cls2_b"])[None, :]
    fw = [params[f"fp{i}_W"] for i in range(3)]
    fb = (params["fp0_b"] + params["fp1_b"] + params["fp2_b"])[None, :]
    mu, logvar, cls, fpo = _mlp1_heads_call(
        x, h1, parts1, eps1, w0, b0, w1, b1, gw, gb, cw, cb, fw, fb,
        params["fd_W1"], params["fd_b1"][None, :],
        params["fd_W2"], params["fd_b2"][None, :])
    adj = _adj_call(mu)
    return adj, mu, logvar, cls, fpo


# speedup vs baseline: 7.9260x; 1.0589x over previous
import functools

import numpy as np
import jax
import jax.numpy as jnp
from jax import lax
from jax.experimental import pallas as pl
from jax.experimental.pallas import tpu as pltpu
from jax.experimental.pallas import tpu_sc as plsc

_N = 4096
_E = 65536
_HID = 128
_BN_S = float(1.0 / np.sqrt(1.0 + 1e-5))

_NC = 2
_NS = 16
_NW = _NC * _NS
_EPT = _E // _NW
_CH = 128
_NCHUNK = _EPT // _CH
_RPT = _N // _NS

_sc_mesh = plsc.VectorSubcoreMesh(core_axis_name="c", subcore_axis_name="s")


_NBUF = 4


@functools.partial(
    pl.kernel,
    out_type=jax.ShapeDtypeStruct((_NC, _N, _HID), jnp.float32),
    mesh=_sc_mesh,
    scratch_types=[
        pltpu.VMEM((_NCHUNK, _CH), jnp.int32),
        pltpu.VMEM((_NCHUNK, _CH), jnp.int32),
        pltpu.VMEM((_NBUF, _CH, _HID), jnp.float32),
        pltpu.VMEM_SHARED((_N, _HID), jnp.float32),
    ] + [pltpu.SemaphoreType.DMA] * _NBUF,
)
def _sc_scatter_add(h_hbm, src_hbm, dst_hbm, out_hbm,
                    src_v, dst_v, stage_v, acc_sh, *gsems):
    cid = lax.axis_index("c")
    sid = lax.axis_index("s")
    wid = sid * _NC + cid
    gidx = pltpu.async_copy(src_hbm.at[wid], src_v, gsems[0])
    didx = pltpu.async_copy(dst_hbm.at[wid], dst_v, gsems[1])
    for r in range(16):
        for c in range(_HID // 16):
            stage_v[_NBUF - 1, r, pl.ds(c * 16, 16)] = (
                jnp.zeros((16,), jnp.float32))
    gidx.wait()
    didx.wait()
    gd = {}
    for j in range(_NBUF - 1):
        gd[j] = pltpu.async_copy(h_hbm.at[src_v.at[j]], stage_v.at[j],
                                 gsems[j])
    row0 = sid * _RPT
    for i in range(_RPT // 16):
        pltpu.sync_copy(stage_v.at[_NBUF - 1, pl.ds(0, 16)],
                        acc_sh.at[pl.ds(row0 + i * 16, 16)])
    gd[_NBUF - 1] = pltpu.async_copy(
        h_hbm.at[src_v.at[_NBUF - 1]], stage_v.at[_NBUF - 1],
        gsems[_NBUF - 1])
    plsc.subcore_barrier()
    for j in range(_NCHUNK):
        b = j % _NBUF
        gd[j].wait()
        pltpu.sync_copy(stage_v.at[b], acc_sh.at[dst_v.at[j]], add=True)
        if j + _NBUF < _NCHUNK:
            gd[j + _NBUF] = pltpu.async_copy(
                h_hbm.at[src_v.at[j + _NBUF]], stage_v.at[b], gsems[b])
    plsc.subcore_barrier()
    pltpu.sync_copy(acc_sh.at[pl.ds(row0, _RPT)],
                    out_hbm.at[cid, pl.ds(row0, _RPT)])


_R = 1024


def _mlp_body(eps_ref, h_ref, p_ref, w0_ref, b0_ref, w1_ref, b1_ref,
              o_ref):
    rst = (1.0 + eps_ref[0]) * h_ref[...] + p_ref[0] + p_ref[1]
    t = jnp.dot(rst, w0_ref[...], preferred_element_type=jnp.float32)
    t = jnp.maximum(t + b0_ref[...], 0.0)
    t = jnp.dot(t, w1_ref[...], preferred_element_type=jnp.float32)
    o_ref[...] = jnp.maximum(t + b1_ref[...], 0.0)


def _mlp_call(h, parts, eps, w0, b0, w1, b1):
    row = pl.BlockSpec((_R, _HID), lambda i: (i, 0))

    def full(shape):
        return pl.BlockSpec(shape, lambda i, _r=len(shape): (0,) * _r)

    return pl.pallas_call(
        _mlp_body,
        grid=(_N // _R,),
        in_specs=[
            pl.BlockSpec(memory_space=pltpu.SMEM),
            row,
            pl.BlockSpec((_NC, _R, _HID), lambda i: (0, i, 0)),
            full((_HID, _HID)), full((1, _HID)),
            full((_HID, _HID)), full((1, _HID)),
        ],
        out_specs=row,
        out_shape=jax.ShapeDtypeStruct((_N, _HID), jnp.float32),
    )(eps, h, parts, w0, b0, w1, b1)


def _mlp1_heads_body(eps_ref, x_ref, h1_ref, p_ref, w0_ref, b0_ref,
                     w1_ref, b1_ref,
                     g0_ref, g1_ref, g2_ref, gb_ref,
                     c0_ref, c1_ref, c2_ref, cb_ref,
                     f0_ref, f1_ref, f2_ref, fb_ref,
                     d1_ref, db1_ref, d2_ref, db2_ref,
                     mu_ref, lv_ref, cls_ref, fpo_ref,
                     a0_ref, a1_ref, a2_ref):
    i = pl.program_id(0)
    dot = functools.partial(jnp.dot, preferred_element_type=jnp.float32)
    x = x_ref[...]
    h1 = h1_ref[...]
    rst = (1.0 + eps_ref[0]) * h1 + p_ref[0] + p_ref[1]
    t = jnp.maximum(dot(rst, w0_ref[...]) + b0_ref[...], 0.0)
    h2 = jnp.maximum(dot(t, w1_ref[...]) + b1_ref[...], 0.0)
    gae = (dot(x, g0_ref[...]) + dot(h1, g1_ref[...])
           + dot(h2, g2_ref[...]) + gb_ref[...])
    mu_ref[...] = gae[:, :_HID]
    lv_ref[...] = gae[:, _HID:]

    s0 = jnp.sum(x, axis=0, keepdims=True)
    s1 = jnp.sum(h1, axis=0, keepdims=True)
    s2 = jnp.sum(h2, axis=0, keepdims=True)

    @pl.when(i == 0)
    def _():
        a0_ref[...] = s0
        a1_ref[...] = s1
        a2_ref[...] = s2

    @pl.when(i > 0)
    def _():
        a0_ref[...] += s0
        a1_ref[...] += s1
        a2_ref[...] += s2

    @pl.when(i == _N // _R - 1)
    def _():
        p0 = a0_ref[...]
        p1 = a1_ref[...]
        p2 = a2_ref[...]
        cls_ref[...] = (dot(p0, c0_ref[...]) + dot(p1, c1_ref[...])
                        + dot(p2, c2_ref[...]) + cb_ref[...])
        fp = (dot(p0, f0_ref[...]) + dot(p1, f1_ref[...])
              + dot(p2, f2_ref[...]) + fb_ref[...])
        tt = jnp.maximum(dot(fp, d1_ref[...]) + db1_ref[...], 0.0)
        fpo_ref[...] = dot(tt, d2_ref[...]) + db2_ref[...]


def _mlp1_heads_call(x, h1, parts, eps, w0, b0, w1, b1,
                     gw, gb, cw, cb, fw, fb, d1, db1, d2, db2):
    row = pl.BlockSpec((_R, _HID), lambda i: (i, 0))

    def full(a):
        return pl.BlockSpec(a.shape, (lambda i, _r=a.ndim: (0,) * _r))

    return pl.pallas_call(
        _mlp1_heads_body,
        grid=(_N // _R,),
        in_specs=[pl.BlockSpec(memory_space=pltpu.SMEM),
                  row, row,
                  pl.BlockSpec((_NC, _R, _HID), lambda i: (0, i, 0))] +
                 [full(a) for a in (w0, b0, w1, b1,
                                    gw[0], gw[1], gw[2], gb,
                                    cw[0], cw[1], cw[2], cb,
                                    fw[0], fw[1], fw[2], fb,
                                    d1, db1, d2, db2)],
        out_specs=[
            row, row,
            pl.BlockSpec((1, 16), lambda i: (0, 0)),
            pl.BlockSpec((1, 2048), lambda i: (0, 0)),
        ],
        out_shape=[
            jax.ShapeDtypeStruct((_N, _HID), jnp.float32),
            jax.ShapeDtypeStruct((_N, _HID), jnp.float32),
            jax.ShapeDtypeStruct((1, 16), jnp.float32),
            jax.ShapeDtypeStruct((1, 2048), jnp.float32),
        ],
        scratch_shapes=[pltpu.VMEM((1, _HID), jnp.float32)] * 3,
    )(eps, x, h1, parts, w0, b0, w1, b1,
      gw[0], gw[1], gw[2], gb, cw[0], cw[1], cw[2], cb,
      fw[0], fw[1], fw[2], fb, d1, db1, d2, db2)


def _adj_body(mu_i_ref, mu_ref, o_ref):
    o_ref[...] = lax.dot_general(
        mu_i_ref[...], mu_ref[...], (((1,), (1,)), ((), ())),
        preferred_element_type=jnp.float32)


def _adj_call(mu):
    return pl.pallas_call(
        _adj_body,
        grid=(_N // _R,),
        in_specs=[
            pl.BlockSpec((_R, _HID), lambda i: (i, 0)),
            pl.BlockSpec((_N, _HID), lambda i: (0, 0)),
        ],
        out_specs=pl.BlockSpec((_R, _N), lambda i: (i, 0)),
        out_shape=jax.ShapeDtypeStruct((_N, _N), jnp.float32),
    )(mu, mu)


def kernel(x, edge_index, params):
    src = edge_index[0].reshape(_NW, _NCHUNK, _CH)
    dst = edge_index[1].reshape(_NW, _NCHUNK, _CH)

    def fold(w, b, g, bb):
        scale = g * _BN_S
        return w * scale[None, :], (b * scale + bb)[None, :]

    def layer_weights(l):
        w0, b0 = fold(params[f"gin{l}_W0"], params[f"gin{l}_b0"],
                      params[f"gin{l}_bn0_g"], params[f"gin{l}_bn0_b"])
        w1, b1 = fold(params[f"gin{l}_W1"], params[f"gin{l}_b1"],
                      params[f"gin{l}_bna_g"], params[f"gin{l}_bna_b"])
        return w0, b0, w1, b1, params[f"gin{l}_eps"].reshape(1)

    parts0 = _sc_scatter_add(x, src, dst)
    w0, b0, w1, b1, eps0 = layer_weights(0)
    h1 = _mlp_call(x, parts0, eps0, w0, b0, w1, b1)

    parts1 = _sc_scatter_add(h1, src, dst)
    w0, b0, w1, b1, eps1 = layer_weights(1)

    gw = [params[f"gae{i}_W"] for i in range(3)]
    gb = (params["gae0_b"] + params["gae1_b"] + params["gae2_b"])[None, :]
    cw = [params[f"cls{i}_W"] for i in range(3)]
    cb = (params["cls0_b"] + params["cls1_b"] + params["cls2_b"])[None, :]
    fw = [params[f"fp{i}_W"] for i in range(3)]
    fb = (params["fp0_b"] + params["fp1_b"] + params["fp2_b"])[None, :]
    mu, logvar, cls, fpo = _mlp1_heads_call(
        x, h1, parts1, eps1, w0, b0, w1, b1, gw, gb, cw, cb, fw, fb,
        params["fd_W1"], params["fd_b1"][None, :],
        params["fd_W2"], params["fd_b2"][None, :])
    adj = _adj_call(mu)
    return adj, mu, logvar, cls, fpo


# speedup vs baseline: 7.9501x; 1.0030x over previous
import functools

import numpy as np
import jax
import jax.numpy as jnp
from jax import lax
from jax.experimental import pallas as pl
from jax.experimental.pallas import tpu as pltpu
from jax.experimental.pallas import tpu_sc as plsc

_N = 4096
_E = 65536
_HID = 128
_BN_S = float(1.0 / np.sqrt(1.0 + 1e-5))

_NC = 2
_NS = 16
_NW = _NC * _NS
_EPT = _E // _NW
_CH = 128
_NCHUNK = _EPT // _CH
_RPT = _N // _NS

_sc_mesh = plsc.VectorSubcoreMesh(core_axis_name="c", subcore_axis_name="s")


_NBUF = 4


@functools.partial(
    pl.kernel,
    out_type=jax.ShapeDtypeStruct((_NC, _N, _HID), jnp.float32),
    mesh=_sc_mesh,
    scratch_types=[
        pltpu.VMEM((_NCHUNK, _CH), jnp.int32),
        pltpu.VMEM((_NCHUNK, _CH), jnp.int32),
        pltpu.VMEM((_NBUF, _CH, _HID), jnp.float32),
        pltpu.VMEM_SHARED((_N, _HID), jnp.float32),
    ] + [pltpu.SemaphoreType.DMA] * _NBUF,
)
def _sc_scatter_add(h_hbm, src_hbm, dst_hbm, out_hbm,
                    src_v, dst_v, stage_v, acc_sh, *gsems):
    cid = lax.axis_index("c")
    sid = lax.axis_index("s")
    wid = sid * _NC + cid
    gidx = pltpu.async_copy(src_hbm.at[wid], src_v, gsems[0])
    didx = pltpu.async_copy(dst_hbm.at[wid], dst_v, gsems[1])
    for r in range(16):
        for c in range(_HID // 16):
            stage_v[_NBUF - 1, r, pl.ds(c * 16, 16)] = (
                jnp.zeros((16,), jnp.float32))
    gidx.wait()
    didx.wait()
    gd = {}
    for j in range(_NBUF - 1):
        gd[j] = pltpu.async_copy(h_hbm.at[src_v.at[j]], stage_v.at[j],
                                 gsems[j])
    row0 = sid * _RPT
    for i in range(_RPT // 16):
        pltpu.sync_copy(stage_v.at[_NBUF - 1, pl.ds(0, 16)],
                        acc_sh.at[pl.ds(row0 + i * 16, 16)])
    gd[_NBUF - 1] = pltpu.async_copy(
        h_hbm.at[src_v.at[_NBUF - 1]], stage_v.at[_NBUF - 1],
        gsems[_NBUF - 1])
    plsc.subcore_barrier()
    for j in range(_NCHUNK):
        b = j % _NBUF
        gd[j].wait()
        pltpu.sync_copy(stage_v.at[b], acc_sh.at[dst_v.at[j]], add=True)
        if j + _NBUF < _NCHUNK:
            gd[j + _NBUF] = pltpu.async_copy(
                h_hbm.at[src_v.at[j + _NBUF]], stage_v.at[b], gsems[b])
    plsc.subcore_barrier()
    pltpu.sync_copy(acc_sh.at[pl.ds(row0, _RPT)],
                    out_hbm.at[cid, pl.ds(row0, _RPT)])


_R = 1024


def _mlp_body(eps_ref, h_ref, p_ref, w0_ref, b0_ref, w1_ref, b1_ref,
              o_ref):
    rst = (1.0 + eps_ref[0]) * h_ref[...] + p_ref[0] + p_ref[1]
    t = jnp.dot(rst, w0_ref[...], preferred_element_type=jnp.float32)
    t = jnp.maximum(t + b0_ref[...], 0.0)
    t = jnp.dot(t, w1_ref[...], preferred_element_type=jnp.float32)
    o_ref[...] = jnp.maximum(t + b1_ref[...], 0.0)


def _mlp_call(h, parts, eps, w0, b0, w1, b1):
    row = pl.BlockSpec((_R, _HID), lambda i: (i, 0))

    def full(shape):
        return pl.BlockSpec(shape, lambda i, _r=len(shape): (0,) * _r)

    return pl.pallas_call(
        _mlp_body,
        grid=(_N // _R,),
        in_specs=[
            pl.BlockSpec(memory_space=pltpu.SMEM),
            row,
            pl.BlockSpec((_NC, _R, _HID), lambda i: (0, i, 0)),
            full((_HID, _HID)), full((1, _HID)),
            full((_HID, _HID)), full((1, _HID)),
        ],
        out_specs=row,
        out_shape=jax.ShapeDtypeStruct((_N, _HID), jnp.float32),
    )(eps, h, parts, w0, b0, w1, b1)


def _mlp1_heads_body(eps_ref, x_ref, h1_ref, p_ref, w0_ref, b0_ref,
                     w1_ref, b1_ref,
                     g0_ref, g1_ref, g2_ref, gb_ref,
                     c0_ref, c1_ref, c2_ref, cb_ref,
                     f0_ref, f1_ref, f2_ref, fb_ref,
                     d1_ref, db1_ref, d2_ref, db2_ref,
                     mu_ref, mubf_ref, lv_ref, cls_ref, fpo_ref,
                     a0_ref, a1_ref, a2_ref):
    i = pl.program_id(0)
    dot = functools.partial(jnp.dot, preferred_element_type=jnp.float32)
    x = x_ref[...]
    h1 = h1_ref[...]
    rst = (1.0 + eps_ref[0]) * h1 + p_ref[0] + p_ref[1]
    t = jnp.maximum(dot(rst, w0_ref[...]) + b0_ref[...], 0.0)
    h2 = jnp.maximum(dot(t, w1_ref[...]) + b1_ref[...], 0.0)
    gae = (dot(x, g0_ref[...]) + dot(h1, g1_ref[...])
           + dot(h2, g2_ref[...]) + gb_ref[...])
    mu = gae[:, :_HID]
    mu_ref[...] = mu
    mubf_ref[...] = mu.astype(jnp.bfloat16)
    lv_ref[...] = gae[:, _HID:]

    s0 = jnp.sum(x, axis=0, keepdims=True)
    s1 = jnp.sum(h1, axis=0, keepdims=True)
    s2 = jnp.sum(h2, axis=0, keepdims=True)

    @pl.when(i == 0)
    def _():
        a0_ref[...] = s0
        a1_ref[...] = s1
        a2_ref[...] = s2

    @pl.when(i > 0)
    def _():
        a0_ref[...] += s0
        a1_ref[...] += s1
        a2_ref[...] += s2

    @pl.when(i == _N // _R - 1)
    def _():
        p0 = a0_ref[...]
        p1 = a1_ref[...]
        p2 = a2_ref[...]
        cls_ref[...] = (dot(p0, c0_ref[...]) + dot(p1, c1_ref[...])
                        + dot(p2, c2_ref[...]) + cb_ref[...])
        fp = (dot(p0, f0_ref[...]) + dot(p1, f1_ref[...])
              + dot(p2, f2_ref[...]) + fb_ref[...])
        tt = jnp.maximum(dot(fp, d1_ref[...]) + db1_ref[...], 0.0)
        fpo_ref[...] = dot(tt, d2_ref[...]) + db2_ref[...]


def _mlp1_heads_call(x, h1, parts, eps, w0, b0, w1, b1,
                     gw, gb, cw, cb, fw, fb, d1, db1, d2, db2):
    row = pl.BlockSpec((_R, _HID), lambda i: (i, 0))

    def full(a):
        return pl.BlockSpec(a.shape, (lambda i, _r=a.ndim: (0,) * _r))

    return pl.pallas_call(
        _mlp1_heads_body,
        grid=(_N // _R,),
        in_specs=[pl.BlockSpec(memory_space=pltpu.SMEM),
                  row, row,
                  pl.BlockSpec((_NC, _R, _HID), lambda i: (0, i, 0))] +
                 [full(a) for a in (w0, b0, w1, b1,
                                    gw[0], gw[1], gw[2], gb,
                                    cw[0], cw[1], cw[2], cb,
                                    fw[0], fw[1], fw[2], fb,
                                    d1, db1, d2, db2)],
        out_specs=[
            row, row, row,
            pl.BlockSpec((1, 16), lambda i: (0, 0)),
            pl.BlockSpec((1, 2048), lambda i: (0, 0)),
        ],
        out_shape=[
            jax.ShapeDtypeStruct((_N, _HID), jnp.float32),
            jax.ShapeDtypeStruct((_N, _HID), jnp.bfloat16),
            jax.ShapeDtypeStruct((_N, _HID), jnp.float32),
            jax.ShapeDtypeStruct((1, 16), jnp.float32),
            jax.ShapeDtypeStruct((1, 2048), jnp.float32),
        ],
        scratch_shapes=[pltpu.VMEM((1, _HID), jnp.float32)] * 3,
    )(eps, x, h1, parts, w0, b0, w1, b1,
      gw[0], gw[1], gw[2], gb, cw[0], cw[1], cw[2], cb,
      fw[0], fw[1], fw[2], fb, d1, db1, d2, db2)


def _adj_body(mu_i_ref, mu_ref, o_ref):
    o_ref[...] = lax.dot_general(
        mu_i_ref[...], mu_ref[...], (((1,), (1,)), ((), ())),
        preferred_element_type=jnp.float32)


def _adj_call(mubf):
    return pl.pallas_call(
        _adj_body,
        grid=(_N // _R,),
        in_specs=[
            pl.BlockSpec((_R, _HID), lambda i: (i, 0)),
            pl.BlockSpec((_N, _HID), lambda i: (0, 0)),
        ],
        out_specs=pl.BlockSpec((_R, _N), lambda i: (i, 0)),
        out_shape=jax.ShapeDtypeStruct((_N, _N), jnp.float32),
    )(mubf, mubf)


def kernel(x, edge_index, params):
    src = edge_index[0].reshape(_NW, _NCHUNK, _CH)
    dst = edge_index[1].reshape(_NW, _NCHUNK, _CH)

    def fold(w, b, g, bb):
        scale = g * _BN_S
        return w * scale[None, :], (b * scale + bb)[None, :]

    def layer_weights(l):
        w0, b0 = fold(params[f"gin{l}_W0"], params[f"gin{l}_b0"],
                      params[f"gin{l}_bn0_g"], params[f"gin{l}_bn0_b"])
        w1, b1 = fold(params[f"gin{l}_W1"], params[f"gin{l}_b1"],
                      params[f"gin{l}_bna_g"], params[f"gin{l}_bna_b"])
        return w0, b0, w1, b1, params[f"gin{l}_eps"].reshape(1)

    parts0 = _sc_scatter_add(x, src, dst)
    w0, b0, w1, b1, eps0 = layer_weights(0)
    h1 = _mlp_call(x, parts0, eps0, w0, b0, w1, b1)

    parts1 = _sc_scatter_add(h1, src, dst)
    w0, b0, w1, b1, eps1 = layer_weights(1)

    gw = [params[f"gae{i}_W"] for i in range(3)]
    gb = (params["gae0_b"] + params["gae1_b"] + params["gae2_b"])[None, :]
    cw = [params[f"cls{i}_W"] for i in range(3)]
    cb = (params["cls0_b"] + params["cls1_b"] + params["cls2_b"])[None, :]
    fw = [params[f"fp{i}_W"] for i in range(3)]
    fb = (params["fp0_b"] + params["fp1_b"] + params["fp2_b"])[None, :]
    mu, mubf, logvar, cls, fpo = _mlp1_heads_call(
        x, h1, parts1, eps1, w0, b0, w1, b1, gw, gb, cw, cb, fw, fb,
        params["fd_W1"], params["fd_b1"][None, :],
        params["fd_W2"], params["fd_b2"][None, :])
    adj = _adj_call(mubf)
    return adj, mu, logvar, cls, fpo


# speedup vs baseline: 8.1035x; 1.0193x over previous
import functools

import numpy as np
import jax
import jax.numpy as jnp
from jax import lax
from jax.experimental import pallas as pl
from jax.experimental.pallas import tpu as pltpu
from jax.experimental.pallas import tpu_sc as plsc

_N = 4096
_E = 65536
_HID = 128
_BN_S = float(1.0 / np.sqrt(1.0 + 1e-5))

_NC = 2
_NS = 16
_NW = _NC * _NS
_EPT = _E // _NW
_CH = 128
_NCHUNK = _EPT // _CH
_RPT = _N // _NS

_sc_mesh = plsc.VectorSubcoreMesh(core_axis_name="c", subcore_axis_name="s")


_NBUF = 4


@functools.partial(
    pl.kernel,
    out_type=jax.ShapeDtypeStruct((_NC, _N, _HID), jnp.float32),
    mesh=_sc_mesh,
    scratch_types=[
        pltpu.VMEM((_NCHUNK, _CH), jnp.int32),
        pltpu.VMEM((_NCHUNK, _CH), jnp.int32),
        pltpu.VMEM((_NBUF, _CH, _HID), jnp.float32),
        pltpu.VMEM_SHARED((_N, _HID), jnp.float32),
    ] + [pltpu.SemaphoreType.DMA] * _NBUF,
)
def _sc_scatter_add(h_hbm, src_hbm, dst_hbm, out_hbm,
                    src_v, dst_v, stage_v, acc_sh, *gsems):
    cid = lax.axis_index("c")
    sid = lax.axis_index("s")
    wid = sid * _NC + cid
    gidx = pltpu.async_copy(src_hbm.at[wid], src_v, gsems[0])
    didx = pltpu.async_copy(dst_hbm.at[wid], dst_v, gsems[1])
    for r in range(16):
        for c in range(_HID // 16):
            stage_v[_NBUF - 1, r, pl.ds(c * 16, 16)] = (
                jnp.zeros((16,), jnp.float32))
    gidx.wait()
    didx.wait()
    gd = {}
    for j in range(_NBUF - 1):
        gd[j] = pltpu.async_copy(h_hbm.at[src_v.at[j]], stage_v.at[j],
                                 gsems[j])
    row0 = sid * _RPT
    for i in range(_RPT // 16):
        pltpu.sync_copy(stage_v.at[_NBUF - 1, pl.ds(0, 16)],
                        acc_sh.at[pl.ds(row0 + i * 16, 16)])
    gd[_NBUF - 1] = pltpu.async_copy(
        h_hbm.at[src_v.at[_NBUF - 1]], stage_v.at[_NBUF - 1],
        gsems[_NBUF - 1])
    plsc.subcore_barrier()
    for j in range(_NCHUNK):
        b = j % _NBUF
        gd[j].wait()
        pltpu.sync_copy(stage_v.at[b], acc_sh.at[dst_v.at[j]], add=True)
        if j + _NBUF < _NCHUNK:
            gd[j + _NBUF] = pltpu.async_copy(
                h_hbm.at[src_v.at[j + _NBUF]], stage_v.at[b], gsems[b])
    plsc.subcore_barrier()
    pltpu.sync_copy(acc_sh.at[pl.ds(row0, _RPT)],
                    out_hbm.at[cid, pl.ds(row0, _RPT)])


_R = 1024


def _mlp_body(eps_ref, h_ref, p_ref, w0_ref, b0_ref, w1_ref, b1_ref,
              o_ref):
    rst = (1.0 + eps_ref[0]) * h_ref[...] + p_ref[0] + p_ref[1]
    t = jnp.dot(rst, w0_ref[...], preferred_element_type=jnp.float32)
    t = jnp.maximum(t + b0_ref[...], 0.0)
    t = jnp.dot(t, w1_ref[...], preferred_element_type=jnp.float32)
    o_ref[...] = jnp.maximum(t + b1_ref[...], 0.0)


def _mlp_call(h, parts, eps, w0, b0, w1, b1):
    row = pl.BlockSpec((_R, _HID), lambda i: (i, 0))

    def full(shape):
        return pl.BlockSpec(shape, lambda i, _r=len(shape): (0,) * _r)

    return pl.pallas_call(
        _mlp_body,
        grid=(_N // _R,),
        in_specs=[
            pl.BlockSpec(memory_space=pltpu.SMEM),
            row,
            pl.BlockSpec((_NC, _R, _HID), lambda i: (0, i, 0)),
            full((_HID, _HID)), full((1, _HID)),
            full((_HID, _HID)), full((1, _HID)),
        ],
        out_specs=row,
        out_shape=jax.ShapeDtypeStruct((_N, _HID), jnp.float32),
    )(eps, h, parts, w0, b0, w1, b1)


def _mlp1_heads_body(eps_ref, x_ref, h1_ref, p_ref, w0_ref, b0_ref,
                     w1_ref, b1_ref,
                     g0_ref, g1_ref, g2_ref, gb_ref,
                     c0_ref, c1_ref, c2_ref, cb_ref,
                     f0_ref, f1_ref, f2_ref, fb_ref,
                     mu_ref, mubf_ref, lv_ref, cls_ref, fp_ref,
                     a0_ref, a1_ref, a2_ref):
    i = pl.program_id(0)
    dot = functools.partial(jnp.dot, preferred_element_type=jnp.float32)
    x = x_ref[...]
    h1 = h1_ref[...]
    rst = (1.0 + eps_ref[0]) * h1 + p_ref[0] + p_ref[1]
    t = jnp.maximum(dot(rst, w0_ref[...]) + b0_ref[...], 0.0)
    h2 = jnp.maximum(dot(t, w1_ref[...]) + b1_ref[...], 0.0)
    gae = (dot(x, g0_ref[...]) + dot(h1, g1_ref[...])
           + dot(h2, g2_ref[...]) + gb_ref[...])
    mu = gae[:, :_HID]
    mu_ref[...] = mu
    mubf_ref[...] = mu.astype(jnp.bfloat16)
    lv_ref[...] = gae[:, _HID:]

    s0 = jnp.sum(x, axis=0, keepdims=True)
    s1 = jnp.sum(h1, axis=0, keepdims=True)
    s2 = jnp.sum(h2, axis=0, keepdims=True)

    @pl.when(i == 0)
    def _():
        a0_ref[...] = s0
        a1_ref[...] = s1
        a2_ref[...] = s2

    @pl.when(i > 0)
    def _():
        a0_ref[...] += s0
        a1_ref[...] += s1
        a2_ref[...] += s2

    @pl.when(i == _N // _R - 1)
    def _():
        p0 = a0_ref[...]
        p1 = a1_ref[...]
        p2 = a2_ref[...]
        cls_ref[...] = (dot(p0, c0_ref[...]) + dot(p1, c1_ref[...])
                        + dot(p2, c2_ref[...]) + cb_ref[...])
        fp_ref[...] = (dot(p0, f0_ref[...]) + dot(p1, f1_ref[...])
                       + dot(p2, f2_ref[...]) + fb_ref[...])


def _mlp1_heads_call(x, h1, parts, eps, w0, b0, w1, b1,
                     gw, gb, cw, cb, fw, fb):
    row = pl.BlockSpec((_R, _HID), lambda i: (i, 0))

    def full(a):
        return pl.BlockSpec(a.shape, (lambda i, _r=a.ndim: (0,) * _r))

    return pl.pallas_call(
        _mlp1_heads_body,
        grid=(_N // _R,),
        in_specs=[pl.BlockSpec(memory_space=pltpu.SMEM),
                  row, row,
                  pl.BlockSpec((_NC, _R, _HID), lambda i: (0, i, 0))] +
                 [full(a) for a in (w0, b0, w1, b1,
                                    gw[0], gw[1], gw[2], gb,
                                    cw[0], cw[1], cw[2], cb,
                                    fw[0], fw[1], fw[2], fb)],
        out_specs=[
            row, row, row,
            pl.BlockSpec((1, 16), lambda i: (0, 0)),
            pl.BlockSpec((1, _HID), lambda i: (0, 0)),
        ],
        out_shape=[
            jax.ShapeDtypeStruct((_N, _HID), jnp.float32),
            jax.ShapeDtypeStruct((_N, _HID), jnp.bfloat16),
            jax.ShapeDtypeStruct((_N, _HID), jnp.float32),
            jax.ShapeDtypeStruct((1, 16), jnp.float32),
            jax.ShapeDtypeStruct((1, _HID), jnp.float32),
        ],
        scratch_shapes=[pltpu.VMEM((1, _HID), jnp.float32)] * 3,
    )(eps, x, h1, parts, w0, b0, w1, b1,
      gw[0], gw[1], gw[2], gb, cw[0], cw[1], cw[2], cb,
      fw[0], fw[1], fw[2], fb)


_NSTEP = _N // _R
_D1C = 1024 // _NSTEP


def _adj_body(mu_i_ref, mu_ref, fp_ref, d1c_ref, db1c_ref, d2c_ref, db2_ref,
              o_ref, fpo_ref, facc_ref):
    i = pl.program_id(0)
    o_ref[...] = lax.dot_general(
        mu_i_ref[...], mu_ref[...], (((1,), (1,)), ((), ())),
        preferred_element_type=jnp.float32)
    z = jnp.maximum(jnp.dot(fp_ref[...], d1c_ref[...],
                            preferred_element_type=jnp.float32)
                    + db1c_ref[...], 0.0)
    contrib = jnp.dot(z, d2c_ref[...], preferred_element_type=jnp.float32)

    @pl.when(i == 0)
    def _():
        facc_ref[...] = contrib

    @pl.when(i > 0)
    def _():
        facc_ref[...] += contrib

    @pl.when(i == _NSTEP - 1)
    def _():
        fpo_ref[...] = facc_ref[...] + db2_ref[...]


def _adj_call(mubf, fp, d1, db1, d2, db2):
    return pl.pallas_call(
        _adj_body,
        grid=(_NSTEP,),
        in_specs=[
            pl.BlockSpec((_R, _HID), lambda i: (i, 0)),
            pl.BlockSpec((_N, _HID), lambda i: (0, 0)),
            pl.BlockSpec((1, _HID), lambda i: (0, 0)),
            pl.BlockSpec((_HID, _D1C), lambda i: (0, i)),
            pl.BlockSpec((1, _D1C), lambda i: (0, i)),
            pl.BlockSpec((_D1C, 2048), lambda i: (i, 0)),
            pl.BlockSpec((1, 2048), lambda i: (0, 0)),
        ],
        out_specs=[
            pl.BlockSpec((_R, _N), lambda i: (i, 0)),
            pl.BlockSpec((1, 2048), lambda i: (0, 0)),
        ],
        out_shape=[
            jax.ShapeDtypeStruct((_N, _N), jnp.float32),
            jax.ShapeDtypeStruct((1, 2048), jnp.float32),
        ],
        scratch_shapes=[pltpu.VMEM((1, 2048), jnp.float32)],
    )(mubf, mubf, fp, d1, db1, d2, db2)


def kernel(x, edge_index, params):
    src = edge_index[0].reshape(_NW, _NCHUNK, _CH)
    dst = edge_index[1].reshape(_NW, _NCHUNK, _CH)

    def fold(w, b, g, bb):
        scale = g * _BN_S
        return w * scale[None, :], (b * scale + bb)[None, :]

    def layer_weights(l):
        w0, b0 = fold(params[f"gin{l}_W0"], params[f"gin{l}_b0"],
                      params[f"gin{l}_bn0_g"], params[f"gin{l}_bn0_b"])
        w1, b1 = fold(params[f"gin{l}_W1"], params[f"gin{l}_b1"],
                      params[f"gin{l}_bna_g"], params[f"gin{l}_bna_b"])
        return w0, b0, w1, b1, params[f"gin{l}_eps"].reshape(1)

    parts0 = _sc_scatter_add(x, src, dst)
    w0, b0, w1, b1, eps0 = layer_weights(0)
    h1 = _mlp_call(x, parts0, eps0, w0, b0, w1, b1)

    parts1 = _sc_scatter_add(h1, src, dst)
    w0, b0, w1, b1, eps1 = layer_weights(1)

    gw = [params[f"gae{i}_W"] for i in range(3)]
    gb = (params["gae0_b"] + params["gae1_b"] + params["gae2_b"])[None, :]
    cw = [params[f"cls{i}_W"] for i in range(3)]
    cb = (params["cls0_b"] + params["cls1_b"] + params["cls2_b"])[None, :]
    fw = [params[f"fp{i}_W"] for i in range(3)]
    fb = (params["fp0_b"] + params["fp1_b"] + params["fp2_b"])[None, :]
    mu, mubf, logvar, cls, fp = _mlp1_heads_call(
        x, h1, parts1, eps1, w0, b0, w1, b1, gw, gb, cw, cb, fw, fb)
    adj, fpo = _adj_call(
        mubf, fp, params["fd_W1"], params["fd_b1"][None, :],
        params["fd_W2"], params["fd_b2"][None, :])
    return adj, mu, logvar, cls, fpo


# speedup vs baseline: 8.1374x; 1.0042x over previous
import functools

import numpy as np
import jax
import jax.numpy as jnp
from jax import lax
from jax.experimental import pallas as pl
from jax.experimental.pallas import tpu as pltpu
from jax.experimental.pallas import tpu_sc as plsc

_N = 4096
_E = 65536
_HID = 128
_BN_S = float(1.0 / np.sqrt(1.0 + 1e-5))

_NC = 2
_NS = 16
_NW = _NC * _NS
_EPT = _E // _NW
_CH = 128
_NCHUNK = _EPT // _CH
_RPT = _N // _NS

_sc_mesh = plsc.VectorSubcoreMesh(core_axis_name="c", subcore_axis_name="s")


_NBUF = 4


@functools.partial(
    pl.kernel,
    out_type=jax.ShapeDtypeStruct((_NC, _N, _HID), jnp.float32),
    mesh=_sc_mesh,
    scratch_types=[
        pltpu.VMEM((_NCHUNK, _CH), jnp.int32),
        pltpu.VMEM((_NCHUNK, _CH), jnp.int32),
        pltpu.VMEM((_NBUF, _CH, _HID), jnp.float32),
        pltpu.VMEM_SHARED((_N, _HID), jnp.float32),
    ] + [pltpu.SemaphoreType.DMA] * _NBUF,
)
def _sc_scatter_add(h_hbm, src_hbm, dst_hbm, out_hbm,
                    src_v, dst_v, stage_v, acc_sh, *gsems):
    cid = lax.axis_index("c")
    sid = lax.axis_index("s")
    wid = sid * _NC + cid
    gidx = pltpu.async_copy(src_hbm.at[wid], src_v, gsems[0])
    didx = pltpu.async_copy(dst_hbm.at[wid], dst_v, gsems[1])
    for r in range(16):
        for c in range(_HID // 16):
            stage_v[_NBUF - 1, r, pl.ds(c * 16, 16)] = (
                jnp.zeros((16,), jnp.float32))
    gidx.wait()
    didx.wait()
    gd = {}
    for j in range(_NBUF - 1):
        gd[j] = pltpu.async_copy(h_hbm.at[src_v.at[j]], stage_v.at[j],
                                 gsems[j])
    row0 = sid * _RPT
    for i in range(_RPT // 16):
        pltpu.sync_copy(stage_v.at[_NBUF - 1, pl.ds(0, 16)],
                        acc_sh.at[pl.ds(row0 + i * 16, 16)])
    gd[_NBUF - 1] = pltpu.async_copy(
        h_hbm.at[src_v.at[_NBUF - 1]], stage_v.at[_NBUF - 1],
        gsems[_NBUF - 1])
    plsc.subcore_barrier()
    for j in range(_NCHUNK):
        b = j % _NBUF
        gd[j].wait()
        pltpu.sync_copy(stage_v.at[b], acc_sh.at[dst_v.at[j]], add=True)
        if j + _NBUF < _NCHUNK:
            gd[j + _NBUF] = pltpu.async_copy(
                h_hbm.at[src_v.at[j + _NBUF]], stage_v.at[b], gsems[b])
    plsc.subcore_barrier()
    pltpu.sync_copy(acc_sh.at[pl.ds(row0, _RPT)],
                    out_hbm.at[cid, pl.ds(row0, _RPT)])


_R = 1024


def _fold(w_ref, b_ref, g_ref, bb_ref):
    scale = g_ref[...] * _BN_S
    return w_ref[...] * scale, b_ref[...] * scale + bb_ref[...]


def _mlp_body(eps_ref, h_ref, p_ref, w0_ref, b0_ref, g0_ref, bb0_ref,
              w1_ref, b1_ref, g1_ref, bb1_ref, o_ref):
    w0, b0 = _fold(w0_ref, b0_ref, g0_ref, bb0_ref)
    w1, b1 = _fold(w1_ref, b1_ref, g1_ref, bb1_ref)
    rst = (1.0 + eps_ref[0]) * h_ref[...] + p_ref[0] + p_ref[1]
    t = jnp.dot(rst, w0, preferred_element_type=jnp.float32)
    t = jnp.maximum(t + b0, 0.0)
    t = jnp.dot(t, w1, preferred_element_type=jnp.float32)
    o_ref[...] = jnp.maximum(t + b1, 0.0)


def _mlp_call(h, parts, lw):
    row = pl.BlockSpec((_R, _HID), lambda i: (i, 0))

    def full(shape):
        return pl.BlockSpec(shape, lambda i, _r=len(shape): (0,) * _r)

    return pl.pallas_call(
        _mlp_body,
        grid=(_N // _R,),
        in_specs=[
            pl.BlockSpec(memory_space=pltpu.SMEM),
            row,
            pl.BlockSpec((_NC, _R, _HID), lambda i: (0, i, 0)),
            full((_HID, _HID)), full((1, _HID)), full((1, _HID)),
            full((1, _HID)),
            full((_HID, _HID)), full((1, _HID)), full((1, _HID)),
            full((1, _HID)),
        ],
        out_specs=row,
        out_shape=jax.ShapeDtypeStruct((_N, _HID), jnp.float32),
    )(lw["eps"], h, parts, *lw["w"])


def _mlp1_heads_body(eps_ref, x_ref, h1_ref, p_ref,
                     w0_ref, b0_ref, bg0_ref, bb0_ref,
                     w1_ref, b1_ref, bg1_ref, bb1_ref,
                     g0_ref, g1_ref, g2_ref, gb_ref,
                     c0_ref, c1_ref, c2_ref, cb_ref,
                     f0_ref, f1_ref, f2_ref, fb_ref,
                     mu_ref, mubf_ref, lv_ref, cls_ref, fp_ref,
                     a0_ref, a1_ref, a2_ref):
    i = pl.program_id(0)
    dot = functools.partial(jnp.dot, preferred_element_type=jnp.float32)
    x = x_ref[...]
    h1 = h1_ref[...]
    w0, b0 = _fold(w0_ref, b0_ref, bg0_ref, bb0_ref)
    w1, b1 = _fold(w1_ref, b1_ref, bg1_ref, bb1_ref)
    rst = (1.0 + eps_ref[0]) * h1 + p_ref[0] + p_ref[1]
    t = jnp.maximum(dot(rst, w0) + b0, 0.0)
    h2 = jnp.maximum(dot(t, w1) + b1, 0.0)
    gae = (dot(x, g0_ref[...]) + dot(h1, g1_ref[...])
           + dot(h2, g2_ref[...]) + gb_ref[...])
    mu = gae[:, :_HID]
    mu_ref[...] = mu
    mubf_ref[...] = mu.astype(jnp.bfloat16)
    lv_ref[...] = gae[:, _HID:]

    s0 = jnp.sum(x, axis=0, keepdims=True)
    s1 = jnp.sum(h1, axis=0, keepdims=True)
    s2 = jnp.sum(h2, axis=0, keepdims=True)

    @pl.when(i == 0)
    def _():
        a0_ref[...] = s0
        a1_ref[...] = s1
        a2_ref[...] = s2

    @pl.when(i > 0)
    def _():
        a0_ref[...] += s0
        a1_ref[...] += s1
        a2_ref[...] += s2

    @pl.when(i == _N // _R - 1)
    def _():
        p0 = a0_ref[...]
        p1 = a1_ref[...]
        p2 = a2_ref[...]
        cls_ref[...] = (dot(p0, c0_ref[...]) + dot(p1, c1_ref[...])
                        + dot(p2, c2_ref[...]) + cb_ref[...])
        fp_ref[...] = (dot(p0, f0_ref[...]) + dot(p1, f1_ref[...])
                       + dot(p2, f2_ref[...]) + fb_ref[...])


def _mlp1_heads_call(x, h1, parts, lw, gw, gb, cw, cb, fw, fb):
    row = pl.BlockSpec((_R, _HID), lambda i: (i, 0))

    def full(a):
        return pl.BlockSpec(a.shape, (lambda i, _r=a.ndim: (0,) * _r))

    return pl.pallas_call(
        _mlp1_heads_body,
        grid=(_N // _R,),
        in_specs=[pl.BlockSpec(memory_space=pltpu.SMEM),
                  row, row,
                  pl.BlockSpec((_NC, _R, _HID), lambda i: (0, i, 0))] +
                 [full(a) for a in lw["w"] +
                  (gw[0], gw[1], gw[2], gb,
                   cw[0], cw[1], cw[2], cb,
                   fw[0], fw[1], fw[2], fb)],
        out_specs=[
            row, row, row,
            pl.BlockSpec((1, 16), lambda i: (0, 0)),
            pl.BlockSpec((1, _HID), lambda i: (0, 0)),
        ],
        out_shape=[
            jax.ShapeDtypeStruct((_N, _HID), jnp.float32),
            jax.ShapeDtypeStruct((_N, _HID), jnp.bfloat16),
            jax.ShapeDtypeStruct((_N, _HID), jnp.float32),
            jax.ShapeDtypeStruct((1, 16), jnp.float32),
            jax.ShapeDtypeStruct((1, _HID), jnp.float32),
        ],
        scratch_shapes=[pltpu.VMEM((1, _HID), jnp.float32)] * 3,
    )(lw["eps"], x, h1, parts, *lw["w"],
      gw[0], gw[1], gw[2], gb, cw[0], cw[1], cw[2], cb,
      fw[0], fw[1], fw[2], fb)


_NSTEP = _N // _R
_D1C = 1024 // _NSTEP


def _adj_body(mu_i_ref, mu_ref, fp_ref, d1c_ref, db1c_ref, d2c_ref, db2_ref,
              o_ref, fpo_ref, facc_ref):
    i = pl.program_id(0)
    o_ref[...] = lax.dot_general(
        mu_i_ref[...], mu_ref[...], (((1,), (1,)), ((), ())),
        preferred_element_type=jnp.float32)
    z = jnp.maximum(jnp.dot(fp_ref[...], d1c_ref[...],
                            preferred_element_type=jnp.float32)
                    + db1c_ref[...], 0.0)
    contrib = jnp.dot(z, d2c_ref[...], preferred_element_type=jnp.float32)

    @pl.when(i == 0)
    def _():
        facc_ref[...] = contrib

    @pl.when(i > 0)
    def _():
        facc_ref[...] += contrib

    @pl.when(i == _NSTEP - 1)
    def _():
        fpo_ref[...] = facc_ref[...] + db2_ref[...]


def _adj_call(mubf, fp, d1, db1, d2, db2):
    return pl.pallas_call(
        _adj_body,
        grid=(_NSTEP,),
        in_specs=[
            pl.BlockSpec((_R, _HID), lambda i: (i, 0)),
            pl.BlockSpec((_N, _HID), lambda i: (0, 0)),
            pl.BlockSpec((1, _HID), lambda i: (0, 0)),
            pl.BlockSpec((_HID, _D1C), lambda i: (0, i)),
            pl.BlockSpec((1, _D1C), lambda i: (0, i)),
            pl.BlockSpec((_D1C, 2048), lambda i: (i, 0)),
            pl.BlockSpec((1, 2048), lambda i: (0, 0)),
        ],
        out_specs=[
            pl.BlockSpec((_R, _N), lambda i: (i, 0)),
            pl.BlockSpec((1, 2048), lambda i: (0, 0)),
        ],
        out_shape=[
            jax.ShapeDtypeStruct((_N, _N), jnp.float32),
            jax.ShapeDtypeStruct((1, 2048), jnp.float32),
        ],
        scratch_shapes=[pltpu.VMEM((1, 2048), jnp.float32)],
    )(mubf, mubf, fp, d1, db1, d2, db2)


def kernel(x, edge_index, params):
    src = edge_index[0].reshape(_NW, _NCHUNK, _CH)
    dst = edge_index[1].reshape(_NW, _NCHUNK, _CH)

    def layer_weights(l):
        return {
            "w": (params[f"gin{l}_W0"], params[f"gin{l}_b0"][None, :],
                  params[f"gin{l}_bn0_g"][None, :],
                  params[f"gin{l}_bn0_b"][None, :],
                  params[f"gin{l}_W1"], params[f"gin{l}_b1"][None, :],
                  params[f"gin{l}_bna_g"][None, :],
                  params[f"gin{l}_bna_b"][None, :]),
            "eps": params[f"gin{l}_eps"].reshape(1),
        }

    parts0 = _sc_scatter_add(x, src, dst)
    h1 = _mlp_call(x, parts0, layer_weights(0))

    parts1 = _sc_scatter_add(h1, src, dst)

    gw = [params[f"gae{i}_W"] for i in range(3)]
    gb = (params["gae0_b"] + params["gae1_b"] + params["gae2_b"])[None, :]
    cw = [params[f"cls{i}_W"] for i in range(3)]
    cb = (params["cls0_b"] + params["cls1_b"] + params["cls2_b"])[None, :]
    fw = [params[f"fp{i}_W"] for i in range(3)]
    fb = (params["fp0_b"] + params["fp1_b"] + params["fp2_b"])[None, :]
    mu, mubf, logvar, cls, fp = _mlp1_heads_call(
        x, h1, parts1, layer_weights(1), gw, gb, cw, cb, fw, fb)
    adj, fpo = _adj_call(
        mubf, fp, params["fd_W1"], params["fd_b1"][None, :],
        params["fd_W2"], params["fd_b2"][None, :])
    return adj, mu, logvar, cls, fpo
